# baseline (speedup 1.0000x reference)
import functools

import jax
import jax.numpy as jnp
from jax import lax
from jax.experimental import pallas as pl
from jax.experimental.pallas import tpu as pltpu
from jax.experimental.pallas import tpu_sc as plsc

VOCAB = 13
EMBED_DIM = 1024
BATCH = 4
SEQ = 8192

_B = BATCH * SEQ
_NW = 32
_BPW = _B // _NW
_C = 32
_NCH = _BPW // _C
_VPAD = 16
_L = 16
_NCOL = EMBED_DIM // _L

_mesh = plsc.VectorSubcoreMesh(core_axis_name="c", subcore_axis_name="s")


@functools.partial(
    pl.kernel,
    mesh=_mesh,
    out_type=jax.ShapeDtypeStruct((_B * EMBED_DIM,), jnp.float32),
    scratch_types=[
        pltpu.VMEM((_VPAD * EMBED_DIM,), jnp.float32),
        pltpu.VMEM((_BPW,), jnp.int32),
        pltpu.VMEM((_C * EMBED_DIM,), jnp.float32),
        pltpu.VMEM((_C * EMBED_DIM,), jnp.float32),
        pltpu.SemaphoreType.DMA,
        pltpu.SemaphoreType.DMA,
    ],
    compiler_params=pltpu.CompilerParams(needs_layout_passes=False),
)
def _emb(x_hbm, table_hbm, out_hbm, table_v, idx_v, rows0, rows1,
         ssem0, ssem1):
    wid = lax.axis_index("s") * 2 + lax.axis_index("c")
    base = wid * _BPW

    pltpu.sync_copy(table_hbm, table_v)
    pltpu.sync_copy(x_hbm.at[pl.ds(base, _BPW)], idx_v)

    rows = (rows0, rows1)
    ssem = (ssem0, ssem1)
    iota = lax.iota(jnp.int32, _L)

    def expand_chunk(b, j):
        buf = rows[b]

        def row_body(r, carry):
            g = j * _C + r
            xi = plsc.load_gather(idx_v, [jnp.full((_L,), g, jnp.int32)])
            src = xi * EMBED_DIM + iota
            for c in range(_NCOL):
                v = plsc.load_gather(table_v, [src + (c * _L)])
                buf[pl.ds(r * EMBED_DIM + c * _L, _L)] = v
            return carry

        lax.fori_loop(0, _C, row_body, 0)

    def s_start(b, j):
        pltpu.async_copy(
            rows[b],
            out_hbm.at[pl.ds((base + j * _C) * EMBED_DIM, _C * EMBED_DIM)],
            ssem[b])

    def s_wait(b, j):
        pltpu.make_async_copy(
            rows[b],
            out_hbm.at[pl.ds((base + j * _C) * EMBED_DIM, _C * EMBED_DIM)],
            ssem[b]).wait()

    expand_chunk(0, 0)
    s_start(0, 0)
    expand_chunk(1, 1)
    s_start(1, 1)

    def body(jj, carry):
        for b, j in ((0, 2 * jj + 2), (1, 2 * jj + 3)):
            s_wait(b, j - 2)
            expand_chunk(b, j)
            s_start(b, j)
        return carry

    lax.fori_loop(0, (_NCH - 2) // 2, body, 0)

    s_wait(0, _NCH - 2)
    s_wait(1, _NCH - 1)


def kernel(x, table):
    table_padded = jnp.pad(table, ((0, _VPAD - VOCAB), (0, 0)))
    out = _emb(x.reshape(_B).astype(jnp.int32), table_padded.reshape(-1))
    return out.reshape(BATCH, SEQ, EMBED_DIM)


# speedup vs baseline: 1.6493x; 1.6493x over previous
import functools

import jax
import jax.numpy as jnp
from jax import lax
from jax.experimental import pallas as pl
from jax.experimental.pallas import tpu as pltpu
from jax.experimental.pallas import tpu_sc as plsc

VOCAB = 13
EMBED_DIM = 1024
BATCH = 4
SEQ = 8192

_B = BATCH * SEQ
_NW = 32
_BPW = _B // _NW
_C = 32
_NCH = _BPW // _C
_VPAD = 16
_L = 16
_NCOL = EMBED_DIM // _L

_mesh = plsc.VectorSubcoreMesh(core_axis_name="c", subcore_axis_name="s")


@functools.partial(
    pl.kernel,
    mesh=_mesh,
    out_type=jax.ShapeDtypeStruct((_B * EMBED_DIM,), jnp.float32),
    scratch_types=[
        pltpu.VMEM((_VPAD * EMBED_DIM,), jnp.float32),
        pltpu.VMEM((_BPW,), jnp.int32),
        pltpu.VMEM((_C * EMBED_DIM,), jnp.float32),
        pltpu.VMEM((_C * EMBED_DIM,), jnp.float32),
        pltpu.SemaphoreType.DMA,
        pltpu.SemaphoreType.DMA,
    ],
    compiler_params=pltpu.CompilerParams(needs_layout_passes=False),
)
def _emb(x_hbm, table_hbm, out_hbm, table_v, idx_v, rows0, rows1,
         ssem0, ssem1):
    wid = lax.axis_index("s") * 2 + lax.axis_index("c")
    base = wid * _BPW

    pltpu.sync_copy(table_hbm, table_v)
    pltpu.sync_copy(x_hbm.at[pl.ds(base, _BPW)], idx_v)

    rows = (rows0, rows1)
    ssem = (ssem0, ssem1)
    iota = lax.iota(jnp.int32, _L)

    def expand_chunk(b, j):
        buf = rows[b]

        @plsc.parallel_loop(0, _C, 1, unroll=2)
        def row_body(r):
            g = j * _C + r
            xi = plsc.load_gather(idx_v, [jnp.full((_L,), g, jnp.int32)])
            src = xi * EMBED_DIM + iota
            for c in range(_NCOL):
                v = plsc.load_gather(table_v, [src + (c * _L)])
                buf[pl.ds(r * EMBED_DIM + c * _L, _L)] = v

    def s_start(b, j):
        pltpu.async_copy(
            rows[b],
            out_hbm.at[pl.ds((base + j * _C) * EMBED_DIM, _C * EMBED_DIM)],
            ssem[b])

    def s_wait(b, j):
        pltpu.make_async_copy(
            rows[b],
            out_hbm.at[pl.ds((base + j * _C) * EMBED_DIM, _C * EMBED_DIM)],
            ssem[b]).wait()

    expand_chunk(0, 0)
    s_start(0, 0)
    expand_chunk(1, 1)
    s_start(1, 1)

    def body(jj, carry):
        for b, j in ((0, 2 * jj + 2), (1, 2 * jj + 3)):
            s_wait(b, j - 2)
            expand_chunk(b, j)
            s_start(b, j)
        return carry

    lax.fori_loop(0, (_NCH - 2) // 2, body, 0)

    s_wait(0, _NCH - 2)
    s_wait(1, _NCH - 1)


def kernel(x, table):
    table_padded = jnp.pad(table, ((0, _VPAD - VOCAB), (0, 0)))
    out = _emb(x.reshape(_B).astype(jnp.int32), table_padded.reshape(-1))
    return out.reshape(BATCH, SEQ, EMBED_DIM)


# speedup vs baseline: 2.9877x; 1.8115x over previous
import functools

import jax
import jax.numpy as jnp
from jax import lax
from jax.experimental import pallas as pl
from jax.experimental.pallas import tpu as pltpu
from jax.experimental.pallas import tpu_sc as plsc

VOCAB = 13
EMBED_DIM = 1024
BATCH = 4
SEQ = 8192

_B = BATCH * SEQ
_NW = 32
_BPW = _B // _NW
_C = 32
_NCH = _BPW // _C
_VPAD = 16
_L = 16

_mesh = plsc.VectorSubcoreMesh(core_axis_name="c", subcore_axis_name="s")


@functools.partial(
    pl.kernel,
    mesh=_mesh,
    out_type=jax.ShapeDtypeStruct((_B, EMBED_DIM), jnp.float32),
    scratch_types=[
        pltpu.VMEM((_BPW,), jnp.int32),
        pltpu.VMEM((_C, EMBED_DIM), jnp.float32),
        pltpu.VMEM((_C, EMBED_DIM), jnp.float32),
        pltpu.SemaphoreType.DMA,
        pltpu.SemaphoreType.DMA,
        pltpu.SemaphoreType.DMA,
        pltpu.SemaphoreType.DMA,
    ],
    compiler_params=pltpu.CompilerParams(needs_layout_passes=False),
)
def _emb(x_hbm, table_hbm, out_hbm, idx_v, rows0, rows1,
         gsem0, gsem1, ssem0, ssem1):
    wid = lax.axis_index("s") * 2 + lax.axis_index("c")
    base = wid * _BPW

    pltpu.sync_copy(x_hbm.at[pl.ds(base, _BPW)], idx_v)

    woff = jnp.full((_L,), wid * _VPAD, jnp.int32)
    for k in range(_BPW // _L):
        sl = pl.ds(k * _L, _L)
        idx_v[sl] = idx_v[sl] + woff

    rows = (rows0, rows1)
    gsem = (gsem0, gsem1)
    ssem = (ssem0, ssem1)

    def g_start(b, j):
        pltpu.async_copy(table_hbm.at[idx_v.at[pl.ds(j * _C, _C)]],
                         rows[b], gsem[b])

    def g_wait(b, j):
        pltpu.make_async_copy(table_hbm.at[idx_v.at[pl.ds(j * _C, _C)]],
                              rows[b], gsem[b]).wait()

    def s_start(b, j):
        pltpu.async_copy(rows[b], out_hbm.at[pl.ds(base + j * _C, _C)],
                         ssem[b])

    def s_wait(b, j):
        pltpu.make_async_copy(rows[b], out_hbm.at[pl.ds(base + j * _C, _C)],
                              ssem[b]).wait()

    g_start(0, 0)
    g_wait(0, 0)
    s_start(0, 0)
    g_start(1, 1)

    def body(jj, carry):
        for b, j in ((1, 2 * jj + 1), (0, 2 * jj + 2)):
            g_wait(b, j)
            s_start(b, j)
            s_wait(1 - b, j - 1)
            g_start(1 - b, j + 1)
        return carry

    lax.fori_loop(0, (_NCH - 2) // 2, body, 0)

    g_wait(1, _NCH - 1)
    s_start(1, _NCH - 1)
    s_wait(0, _NCH - 2)
    s_wait(1, _NCH - 1)


def kernel(x, table):
    table_padded = jnp.pad(table, ((0, _VPAD - VOCAB), (0, 0)))
    table_rep = jnp.tile(table_padded, (_NW, 1))
    out = _emb(x.reshape(_B).astype(jnp.int32), table_rep)
    return out.reshape(BATCH, SEQ, EMBED_DIM)
